# Initial kernel scaffold; baseline (speedup 1.0000x reference)
import jax
import jax.numpy as jnp
from jax.experimental import pallas as pl


def kernel(score, target):
    raise NotImplementedError("write your pallas kernel here")


# speedup vs baseline: 35.0717x; 35.0717x over previous
import functools

import jax
import jax.numpy as jnp
from jax import lax
from jax.experimental import pallas as pl
from jax.experimental.pallas import tpu as pltpu

_THRESH = 0.7
_MIN_KEPT = 100000
_C = 19
_BH = 64


def _stats_body(thr_ref, score_ref, tgt_ref, cnt_lt_ref, sum_lt_ref, cnt_le_ref):
    b = pl.program_id(0)
    h = pl.program_id(1)

    @pl.when((b == 0) & (h == 0))
    def _init():
        cnt_lt_ref[0, 0] = 0.0
        sum_lt_ref[0, 0] = 0.0
        cnt_le_ref[0, 0] = 0.0

    x = score_ref[0]
    t = tgt_ref[0]
    m = jnp.max(x, axis=0)
    onehot = lax.broadcasted_iota(jnp.int32, x.shape, 0) == t[None]
    x_t = jnp.sum(jnp.where(onehot, x, 0.0), axis=0)
    e = jnp.exp(x - m[None])
    s = jnp.sum(e, axis=0)
    e_t = jnp.exp(x_t - m)
    pg = e_t / s
    nll = jnp.log(s) + (m - x_t)
    thr = thr_ref[0, 0]
    lt = pg < thr
    cnt_lt_ref[0, 0] += jnp.sum(lt.astype(jnp.float32))
    sum_lt_ref[0, 0] += jnp.sum(jnp.where(lt, nll, 0.0))
    cnt_le_ref[0, 0] += jnp.sum((pg <= thr).astype(jnp.float32))


def _stats(score, target, thr):
    b, c, hh, w = score.shape
    grid = (b, hh // _BH)
    out = pl.pallas_call(
        _stats_body,
        grid=grid,
        in_specs=[
            pl.BlockSpec(memory_space=pltpu.SMEM),
            pl.BlockSpec((1, c, _BH, w), lambda i, j: (i, 0, j, 0)),
            pl.BlockSpec((1, _BH, w), lambda i, j: (i, j, 0)),
        ],
        out_specs=[
            pl.BlockSpec(memory_space=pltpu.SMEM),
            pl.BlockSpec(memory_space=pltpu.SMEM),
            pl.BlockSpec(memory_space=pltpu.SMEM),
        ],
        out_shape=[jax.ShapeDtypeStruct((1, 1), jnp.float32)] * 3,
    )(thr, score, target)
    return out[0][0, 0], out[1][0, 0], out[2][0, 0]


def kernel(score, target):
    kp1 = jnp.float32(_MIN_KEPT + 1)
    thr0 = jnp.full((1, 1), _THRESH, jnp.float32)
    cnt_lt, sum_lt, cnt_le = _stats(score, target, thr0)

    def case_a(_):
        return sum_lt / jnp.maximum(cnt_lt, 1.0)

    def case_b(_):
        def cond(st):
            lo, hi = st
            return hi - lo > 1

        def body(st):
            lo, hi = st
            mid = (lo + hi) // 2
            t = lax.bitcast_convert_type(mid, jnp.float32).reshape(1, 1)
            _, _, c_le = _stats(score, target, t)
            ge = c_le >= kp1
            return jnp.where(ge, lo, mid), jnp.where(ge, mid, hi)

        lo0 = jnp.int32(0x3F333333)
        hi0 = jnp.int32(0x3F800000)
        _, hi = lax.while_loop(cond, body, (lo0, hi0))
        vstar = lax.bitcast_convert_type(hi, jnp.float32).reshape(1, 1)
        c_lt2, s_lt2, _ = _stats(score, target, vstar)
        return s_lt2 / jnp.maximum(c_lt2, 1.0)

    return lax.cond(cnt_le < kp1, case_b, case_a, None)
